# baseline (speedup 1.0000x reference)
import functools

import jax
import jax.numpy as jnp
from jax import lax
from jax.experimental import pallas as pl
from jax.experimental.pallas import tpu as pltpu
from jax.experimental.pallas import tpu_sc as plsc

B = 16384
K = 64
N = 1000

_INFO = plsc.get_sparse_core_info()
NC = _INFO.num_cores
NS = _INFO.num_subcores
NW = NC * NS
IDX_W = 128
BPW = B // NW
JC = BPW // IDX_W

BM = 1024
NQ = 4
RQ = BM // NQ


def _sc_gather_prod_pairs(idx0, idx1, f0, f1):
    mesh = plsc.VectorSubcoreMesh(core_axis_name="c", subcore_axis_name="s")

    @functools.partial(
        pl.kernel,
        mesh=mesh,
        compiler_params=pltpu.CompilerParams(use_tc_tiling_on_sc=False),
        out_type=jax.ShapeDtypeStruct((B // 2, 2 * K), jnp.float32),
        scratch_types=[
            pltpu.VMEM((JC, IDX_W), jnp.int32),
            pltpu.VMEM((JC, IDX_W), jnp.int32),
            pltpu.VMEM((BPW, K), jnp.float32),
            pltpu.VMEM((BPW, K), jnp.float32),
            pltpu.SemaphoreType.DMA,
            pltpu.SemaphoreType.DMA,
        ],
    )
    def sc_k(idx0_hbm, idx1_hbm, f0_hbm, f1_hbm, out_hbm,
             idx0_v, idx1_v, r0, r1, sem0, sem1):
        wid = lax.axis_index("s") * NC + lax.axis_index("c")
        base = wid * JC
        pltpu.sync_copy(idx0_hbm.at[pl.ds(base, JC)], idx0_v)
        pltpu.sync_copy(idx1_hbm.at[pl.ds(base, JC)], idx1_v)
        copies = []
        for j in range(JC):
            dst = pl.ds(j * IDX_W, IDX_W)
            copies.append(
                pltpu.async_copy(f0_hbm.at[idx0_v.at[j]], r0.at[dst], sem0))
            copies.append(
                pltpu.async_copy(f1_hbm.at[idx1_v.at[j]], r1.at[dst], sem1))
        for c in copies:
            c.wait()

        def body(r, carry):
            for c in range(K // 16):
                s = pl.ds(c * 16, 16)
                r0[r, s] = r0[r, s] * r1[r, s]
            return carry

        lax.fori_loop(0, BPW, body, 0)

        par = lax.rem(wid, 4)
        qbase = (wid // 4) * 1024 + lax.rem(wid, 2) * 512
        h = par // 2
        pltpu.sync_copy(
            r0, out_hbm.at[pl.ds(qbase, BPW), pl.ds(h * K, K)])

    return sc_k(idx0, idx1, f0, f1)


def _tc_matmul_pairs(pairs, f2t_lo, f2t_hi):
    grid = (B // BM,)

    def body(p_ref, lo_ref, hi_ref, out_ref, obuf, sems):
        i = pl.program_id(0)
        nsteps = pl.num_programs(0)

        def slot_copies(step, sbase, do_start):
            for q in range(NQ):
                cp = pltpu.make_async_copy(
                    obuf.at[pl.ds(sbase + q * RQ, RQ), :],
                    out_ref.at[pl.ds(step * BM + q * RQ, RQ), :],
                    sems.at[sbase // BM, q],
                )
                if do_start:
                    cp.start()
                else:
                    cp.wait()

        def for_parity(step, do_start):
            @pl.when(lax.rem(step, 2) == 0)
            def _():
                slot_copies(step, 0, do_start)

            @pl.when(lax.rem(step, 2) == 1)
            def _():
                slot_copies(step, BM, do_start)

        @pl.when(i >= 2)
        def _():
            for_parity(i - 2, False)

        lhs = p_ref[...]

        @pl.when(lax.rem(i, 2) == 0)
        def _():
            obuf[pl.ds(0, BM), :] = jnp.dot(
                lhs, lo_ref[...], preferred_element_type=jnp.float32)

        @pl.when(lax.rem(i, 2) == 1)
        def _():
            obuf[pl.ds(BM, BM), :] = jnp.dot(
                lhs, hi_ref[...], preferred_element_type=jnp.float32)

        for_parity(i, True)

        @pl.when(i == nsteps - 1)
        def _():
            for_parity(i - 1, False)
            for_parity(i, False)

    return pl.pallas_call(
        body,
        grid=grid,
        in_specs=[
            pl.BlockSpec((BM, 2 * K), lambda i: (i // 2, 0)),
            pl.BlockSpec((2 * K, N), lambda i: (0, 0)),
            pl.BlockSpec((2 * K, N), lambda i: (0, 0)),
        ],
        out_specs=pl.BlockSpec(memory_space=pl.ANY),
        out_shape=jax.ShapeDtypeStruct((B, N), jnp.float32),
        scratch_shapes=[
            pltpu.VMEM((2 * BM, N), jnp.float32),
            pltpu.SemaphoreType.DMA((2, NQ)),
        ],
    )(pairs, f2t_lo, f2t_hi)


def _sig_clip(log_sigma):
    def sig_body(ls_ref, sig_ref):
        sig_ref[...] = jnp.clip(ls_ref[...], -2.5, 0.0)

    return pl.pallas_call(
        sig_body,
        out_shape=jax.ShapeDtypeStruct((1, N), jnp.float32),
    )(log_sigma)


def kernel(indices, F0, F1, F2, log_sigma):
    idx0 = indices[:, 0].reshape(NW * JC, IDX_W).astype(jnp.int32)
    idx1 = indices[:, 1].reshape(NW * JC, IDX_W).astype(jnp.int32)
    pairs = _sc_gather_prod_pairs(idx0, idx1, F0, F1)
    f2t = F2.T
    zeros = jnp.zeros((K, N), dtype=jnp.float32)
    f2t_lo = jnp.concatenate([f2t, zeros], axis=0)
    f2t_hi = jnp.concatenate([zeros, f2t], axis=0)
    res = _tc_matmul_pairs(pairs, f2t_lo, f2t_hi)
    sig = _sig_clip(log_sigma)
    return (res, sig)


# speedup vs baseline: 1.3500x; 1.3500x over previous
import functools

import jax
import jax.numpy as jnp
from jax import lax
from jax.experimental import pallas as pl
from jax.experimental.pallas import tpu as pltpu
from jax.experimental.pallas import tpu_sc as plsc

B = 16384
K = 64
N = 1000

_INFO = plsc.get_sparse_core_info()
NC = _INFO.num_cores
NS = _INFO.num_subcores
NW = NC * NS
IDX_W = 128
BPW = B // NW
JC = BPW // IDX_W

BM = 1024
NQ = 5
RQ = N // NQ


def _sc_gather_prod_pairs(idx0, idx1, f0, f1):
    mesh = plsc.VectorSubcoreMesh(core_axis_name="c", subcore_axis_name="s")

    @functools.partial(
        pl.kernel,
        mesh=mesh,
        compiler_params=pltpu.CompilerParams(use_tc_tiling_on_sc=False),
        out_type=jax.ShapeDtypeStruct((B // 2, 2 * K), jnp.float32),
        scratch_types=[
            pltpu.VMEM((JC, IDX_W), jnp.int32),
            pltpu.VMEM((JC, IDX_W), jnp.int32),
            pltpu.VMEM((BPW, K), jnp.float32),
            pltpu.VMEM((BPW, K), jnp.float32),
            pltpu.SemaphoreType.DMA,
            pltpu.SemaphoreType.DMA,
        ],
    )
    def sc_k(idx0_hbm, idx1_hbm, f0_hbm, f1_hbm, out_hbm,
             idx0_v, idx1_v, r0, r1, sem0, sem1):
        wid = lax.axis_index("s") * NC + lax.axis_index("c")
        base = wid * JC
        pltpu.sync_copy(idx0_hbm.at[pl.ds(base, JC)], idx0_v)
        pltpu.sync_copy(idx1_hbm.at[pl.ds(base, JC)], idx1_v)
        copies = []
        for j in range(JC):
            dst = pl.ds(j * IDX_W, IDX_W)
            copies.append(
                pltpu.async_copy(f0_hbm.at[idx0_v.at[j]], r0.at[dst], sem0))
            copies.append(
                pltpu.async_copy(f1_hbm.at[idx1_v.at[j]], r1.at[dst], sem1))
        for c in copies:
            c.wait()

        def body(r, carry):
            for c in range(K // 16):
                s = pl.ds(c * 16, 16)
                r0[r, s] = r0[r, s] * r1[r, s]
            return carry

        lax.fori_loop(0, BPW, body, 0)

        par = lax.rem(wid, 4)
        qbase = (wid // 4) * 1024 + lax.rem(wid, 2) * 512
        h = par // 2
        pltpu.sync_copy(
            r0, out_hbm.at[pl.ds(qbase, BPW), pl.ds(h * K, K)])

    return sc_k(idx0, idx1, f0, f1)


def _tc_matmul_pairs(pairs, f2_lo, f2_hi):
    grid = (B // BM,)

    def body(p_ref, lo_ref, hi_ref, out_ref, obuf, sems):
        i = pl.program_id(0)
        nsteps = pl.num_programs(0)

        def slot_copies(step, slot, do_start):
            for q in range(NQ):
                cp = pltpu.make_async_copy(
                    obuf.at[pl.ds(slot * N + q * RQ, RQ), :],
                    out_ref.at[pl.ds(q * RQ, RQ), pl.ds(step * BM, BM)],
                    sems.at[slot, q],
                )
                if do_start:
                    cp.start()
                else:
                    cp.wait()

        def for_parity(step, do_start):
            @pl.when(lax.rem(step, 2) == 0)
            def _():
                slot_copies(step, 0, do_start)

            @pl.when(lax.rem(step, 2) == 1)
            def _():
                slot_copies(step, 1, do_start)

        @pl.when(i >= 2)
        def _():
            for_parity(i - 2, False)

        rhs = p_ref[...]
        cdims = (((1,), (1,)), ((), ()))

        @pl.when(lax.rem(i, 2) == 0)
        def _():
            obuf[pl.ds(0, N), :] = lax.dot_general(
                lo_ref[...], rhs, cdims, preferred_element_type=jnp.float32)

        @pl.when(lax.rem(i, 2) == 1)
        def _():
            obuf[pl.ds(N, N), :] = lax.dot_general(
                hi_ref[...], rhs, cdims, preferred_element_type=jnp.float32)

        for_parity(i, True)

        @pl.when(i == nsteps - 1)
        def _():
            for_parity(i - 1, False)
            for_parity(i, False)

    return pl.pallas_call(
        body,
        grid=grid,
        in_specs=[
            pl.BlockSpec((BM, 2 * K), lambda i: (i // 2, 0)),
            pl.BlockSpec((N, 2 * K), lambda i: (0, 0)),
            pl.BlockSpec((N, 2 * K), lambda i: (0, 0)),
        ],
        out_specs=pl.BlockSpec(memory_space=pl.ANY),
        out_shape=jax.ShapeDtypeStruct((N, B), jnp.float32),
        scratch_shapes=[
            pltpu.VMEM((2 * N, BM), jnp.float32),
            pltpu.SemaphoreType.DMA((2, NQ)),
        ],
    )(pairs, f2_lo, f2_hi)


def _sig_clip(log_sigma):
    def sig_body(ls_ref, sig_ref):
        sig_ref[...] = jnp.clip(ls_ref[...], -2.5, 0.0)

    return pl.pallas_call(
        sig_body,
        out_shape=jax.ShapeDtypeStruct((1, N), jnp.float32),
    )(log_sigma)


def kernel(indices, F0, F1, F2, log_sigma):
    idx0 = indices[:, 0].reshape(NW * JC, IDX_W).astype(jnp.int32)
    idx1 = indices[:, 1].reshape(NW * JC, IDX_W).astype(jnp.int32)
    pairs = _sc_gather_prod_pairs(idx0, idx1, F0, F1)
    zeros = jnp.zeros((N, K), dtype=jnp.float32)
    f2_lo = jnp.concatenate([F2, zeros], axis=1)
    f2_hi = jnp.concatenate([zeros, F2], axis=1)
    resT = _tc_matmul_pairs(pairs, f2_lo, f2_hi)
    res = resT.T
    sig = _sig_clip(log_sigma)
    return (res, sig)
